# baseline (speedup 1.0000x reference)
import functools
import numpy as np
import jax
import jax.numpy as jnp
from jax import lax
from jax.experimental import pallas as pl
from jax.experimental.pallas import tpu as pltpu
from jax.experimental.pallas import tpu_sc as plsc

_B, _N, _NC, _NP = 4, 8192, 256, 64
_CB = 64
_PB = 16
_R = _PB * _NP
_NW = 32
_QPW = (_B * _NC) // _NW
_GW = 96
_BISECT = 20


def _grid_np(n_p):
    side = int(np.sqrt(n_p))
    xs = np.linspace(-1.0, 1.0, side, dtype=np.float32)
    gx, gy = np.meshgrid(xs, xs, indexing='ij')
    return np.stack([gx.reshape(-1), gy.reshape(-1)], axis=-1)


_GRID = _grid_np(_NP)


def _fps_body(pts_ref, cen_ref):
    px = pts_ref[0]
    py = pts_ref[1]
    pz = pts_ref[2]
    iota = lax.broadcasted_iota(jnp.int32, (_B, _N), 1)
    iota_c = lax.broadcasted_iota(jnp.int32, (_B, _NC), 1)
    d = (px - px[:, :1]) ** 2 + (py - py[:, :1]) ** 2 + (pz - pz[:, :1]) ** 2
    cenx = jnp.where(iota_c == 0, px[:, :1], 0.0)
    ceny = jnp.where(iota_c == 0, py[:, :1], 0.0)
    cenz = jnp.where(iota_c == 0, pz[:, :1], 0.0)

    def step(t, carry):
        d, cenx, ceny, cenz = carry
        m = jnp.max(d, axis=1, keepdims=True)
        far = jnp.min(jnp.where(d == m, iota, jnp.int32(_N)), axis=1,
                      keepdims=True)
        oh = iota == far
        cx = jnp.sum(jnp.where(oh, px, 0.0), axis=1, keepdims=True)
        cy = jnp.sum(jnp.where(oh, py, 0.0), axis=1, keepdims=True)
        cz = jnp.sum(jnp.where(oh, pz, 0.0), axis=1, keepdims=True)
        dn = (px - cx) ** 2 + (py - cy) ** 2 + (pz - cz) ** 2
        d = jnp.minimum(d, dn)
        sel = iota_c == t
        cenx = jnp.where(sel, cx, cenx)
        ceny = jnp.where(sel, cy, ceny)
        cenz = jnp.where(sel, cz, cenz)
        return d, cenx, ceny, cenz

    d, cenx, ceny, cenz = lax.fori_loop(1, _NC, step, (d, cenx, ceny, cenz))
    cen_ref[0] = cenx
    cen_ref[1] = ceny
    cen_ref[2] = cenz


def _fps(data3):
    return pl.pallas_call(
        _fps_body,
        out_shape=jax.ShapeDtypeStruct((3, _B, _NC), jnp.float32),
    )(data3)


def _kthresh_body(pts_ref, cen_ref, d_ref, t_ref):
    px = pts_ref[0, 0:1, :]
    py = pts_ref[0, 1:2, :]
    pz = pts_ref[0, 2:3, :]
    p3 = pts_ref[0]
    q = cen_ref[0]
    qsq = jnp.sum(q * q, axis=1, keepdims=True)
    psq = px * px + py * py + pz * pz
    qp = lax.dot_general(q, p3, (((1,), (0,)), ((), ())))
    d = qsq + psq - 2.0 * qp
    d_ref[0] = d

    hi = jnp.max(jnp.min(d.reshape(_CB, _NP, _N // _NP), axis=2), axis=1,
                 keepdims=True)
    lo = jnp.zeros_like(hi)

    def it(_, lohi):
        lo, hi = lohi
        mid = 0.5 * (lo + hi)
        cnt = jnp.sum(jnp.where(d <= mid, 1.0, 0.0), axis=1, keepdims=True)
        ge = cnt >= jnp.float32(_NP)
        hi = jnp.where(ge, mid, hi)
        lo = jnp.where(ge, lo, mid)
        return lo, hi

    lo, hi = lax.fori_loop(0, _BISECT, it, (lo, hi))
    t_ref[0] = hi


def _kthresh(data, centers):
    nblk = _NC // _CB
    return pl.pallas_call(
        _kthresh_body,
        grid=(_B, nblk),
        in_specs=[
            pl.BlockSpec((1, 3, _N), lambda b, c: (b, 0, 0)),
            pl.BlockSpec((1, _CB, 3), lambda b, c: (b, c, 0)),
        ],
        out_specs=[
            pl.BlockSpec((1, _CB, _N), lambda b, c: (b, c, 0)),
            pl.BlockSpec((1, _CB, 1), lambda b, c: (b * nblk + c, 0, 0)),
        ],
        out_shape=[
            jax.ShapeDtypeStruct((_B, _NC, _N), jnp.float32),
            jax.ShapeDtypeStruct((_B * nblk, _CB, 1), jnp.float32),
        ],
    )(data, centers)


def _sc_body(d_hbm, t16_hbm, data_hbm, outx_hbm, outy_hbm, outz_hbm, aux_hbm,
             pxb, pyb, pzb, dbuf, tbuf, gxb, gyb, gzb, auxb):
    cid = lax.axis_index("c")
    sid = lax.axis_index("s")
    wid = sid * 2 + cid
    b = wid // (_NW // _B)
    pltpu.sync_copy(data_hbm.at[b * 3], pxb)
    pltpu.sync_copy(data_hbm.at[b * 3 + 1], pyb)
    pltpu.sync_copy(data_hbm.at[b * 3 + 2], pzb)
    woff = pl.multiple_of(wid * (_QPW * 16), 512)
    pltpu.sync_copy(t16_hbm.at[pl.ds(woff, _QPW * 16)], tbuf)
    iota16 = lax.iota(jnp.int32, 16)

    def per_query(i, carry):
        q = wid * _QPW + i
        pltpu.sync_copy(d_hbm.at[q], dbuf)
        tv = tbuf[pl.ds(i * 16, 16)]
        base = i * _GW

        lim = lax.broadcast(base + (_GW - 1), (16,))

        one16 = jnp.ones((16,), jnp.int32)

        def scan_vreg(j, cur):
            v = dbuf[pl.ds(j * 16, 16)]
            m = v <= tv
            ranks = plsc.cumsum(m.astype(jnp.int32))
            pos = jnp.minimum(cur + ranks - one16, lim)
            gx = pxb[pl.ds(j * 16, 16)]
            gy = pyb[pl.ds(j * 16, 16)]
            gz = pzb[pl.ds(j * 16, 16)]
            plsc.store_scatter(gxb, [pos], gx, mask=m)
            plsc.store_scatter(gyb, [pos], gy, mask=m)
            plsc.store_scatter(gzb, [pos], gz, mask=m)
            return cur + plsc.all_reduce_population_count(m)

        cur0 = lax.broadcast(base, (16,))
        plsc.parallel_loop(0, _N // 16, carry=cur0, unroll=8)(scan_vreg)

        bofs = pl.multiple_of(base, 8)
        x0 = gxb[pl.ds(bofs, 16)]
        x1 = gxb[pl.ds(bofs + 16, 16)]
        x2 = gxb[pl.ds(bofs + 32, 16)]
        x3 = gxb[pl.ds(bofs + 48, 16)]
        y0 = gyb[pl.ds(bofs, 16)]
        y1 = gyb[pl.ds(bofs + 16, 16)]
        y2 = gyb[pl.ds(bofs + 32, 16)]
        y3 = gyb[pl.ds(bofs + 48, 16)]
        z0 = gzb[pl.ds(bofs, 16)]
        z1 = gzb[pl.ds(bofs + 16, 16)]
        z2 = gzb[pl.ds(bofs + 32, 16)]
        z3 = gzb[pl.ds(bofs + 48, 16)]
        mxv = lax.broadcast(jnp.sum((x0 + x1) + (x2 + x3)) * (1.0 / _NP), (16,))
        myv = lax.broadcast(jnp.sum((y0 + y1) + (y2 + y3)) * (1.0 / _NP), (16,))
        mzv = lax.broadcast(jnp.sum((z0 + z1) + (z2 + z3)) * (1.0 / _NP), (16,))

        def _nsq(a, b, c):
            da, db, dc = a - mxv, b - myv, c - mzv
            return da * da + db * db + dc * dc

        n0 = _nsq(x0, y0, z0)
        n1 = _nsq(x1, y1, z1)
        n2 = _nsq(x2, y2, z2)
        n3 = _nsq(x3, y3, z3)
        msqv = lax.broadcast(
            jnp.max(jnp.maximum(jnp.maximum(n0, n1), jnp.maximum(n2, n3))),
            (16,))
        zv = jnp.zeros((16,), jnp.float32)
        av = jnp.where(iota16 == 0, mxv,
                       jnp.where(iota16 == 1, myv,
                                 jnp.where(iota16 == 2, mzv,
                                           jnp.where(iota16 == 3, msqv, zv))))
        auxb[pl.ds(i * 16, 16)] = av

        oofs = pl.multiple_of(q * _NP, 64)
        pltpu.sync_copy(gxb.at[pl.ds(bofs, _NP)], outx_hbm.at[pl.ds(oofs, _NP)])
        pltpu.sync_copy(gyb.at[pl.ds(bofs, _NP)], outy_hbm.at[pl.ds(oofs, _NP)])
        pltpu.sync_copy(gzb.at[pl.ds(bofs, _NP)], outz_hbm.at[pl.ds(oofs, _NP)])
        return carry

    lax.fori_loop(0, _QPW, per_query, 0)
    pltpu.sync_copy(auxb, aux_hbm.at[pl.ds(woff, _QPW * 16)])


def _sc_select(d_flat, t16, data):
    nq = _B * _NC
    f = pl.kernel(
        _sc_body,
        mesh=plsc.VectorSubcoreMesh(core_axis_name="c", subcore_axis_name="s"),
        compiler_params=pltpu.CompilerParams(needs_layout_passes=False),
        out_type=[
            jax.ShapeDtypeStruct((nq * _NP,), jnp.float32),
            jax.ShapeDtypeStruct((nq * _NP,), jnp.float32),
            jax.ShapeDtypeStruct((nq * _NP,), jnp.float32),
            jax.ShapeDtypeStruct((nq * 16,), jnp.float32),
        ],
        scratch_types=[
            pltpu.VMEM((_N,), jnp.float32),
            pltpu.VMEM((_N,), jnp.float32),
            pltpu.VMEM((_N,), jnp.float32),
            pltpu.VMEM((_N,), jnp.float32),
            pltpu.VMEM((_QPW * 16,), jnp.float32),
            pltpu.VMEM((_QPW * _GW,), jnp.float32),
            pltpu.VMEM((_QPW * _GW,), jnp.float32),
            pltpu.VMEM((_QPW * _GW,), jnp.float32),
            pltpu.VMEM((_QPW * 16,), jnp.float32),
        ],
    )
    return f(d_flat, t16, data)


def _mm(a, b):
    return lax.dot_general(a, b, (((1,), (0,)), ((), ())),
                           preferred_element_type=jnp.float32)


def _fold_body(xr_ref, auxb_ref, auxf_ref, grid_ref,
               We1_ref, be1_ref, We2_ref, be2_ref, We3_ref, be3_ref,
               W1ac_ref, W1ag_ref, b1a_ref, W1b_ref, b1b_ref, W1c_ref,
               b1c_ref, W2ac_ref, W2af_ref, b2a_ref, W2b_ref, b2b_ref,
               W2c_ref, b2c_ref, out_ref):
    scale = jnp.sqrt(jnp.max(auxf_ref[:, 3:4]))
    inv = 1.0 / scale

    riota = lax.broadcasted_iota(jnp.int32, (_R, _PB), 0)
    piota = lax.broadcasted_iota(jnp.int32, (_R, _PB), 1)
    E = (riota // _NP == piota).astype(jnp.float32)
    riota2 = lax.broadcasted_iota(jnp.int32, (_R, _NP), 0)
    kiota = lax.broadcasted_iota(jnp.int32, (_R, _NP), 1)
    S = (riota2 % _NP == kiota).astype(jnp.float32)

    meanE = _mm(E, auxb_ref[:, 0:3])
    xx = (xr_ref[:, 0:1] - meanE[:, 0:1]) * inv
    yy = (xr_ref[:, 1:2] - meanE[:, 1:2]) * inv
    zz = (xr_ref[:, 2:3] - meanE[:, 2:3]) * inv
    We1 = We1_ref[...]
    h = jnp.maximum(xx * We1[0:1, :] + yy * We1[1:2, :] + zz * We1[2:3, :]
                    + be1_ref[...], 0.0)
    h = jnp.maximum(_mm(h, We2_ref[...]) + be2_ref[...], 0.0)
    h = _mm(h, We3_ref[...]) + be3_ref[...]
    code = jnp.max(h.reshape(_PB, _NP, 128), axis=1)

    g = grid_ref[...]
    W1ag = W1ag_ref[...]
    gW = g[:, 0:1] * W1ag[0:1, :] + g[:, 1:2] * W1ag[1:2, :]
    f = jnp.maximum(_mm(E, _mm(code, W1ac_ref[...])) + _mm(S, gW)
                    + b1a_ref[...], 0.0)
    f = jnp.maximum(_mm(f, W1b_ref[...]) + b1b_ref[...], 0.0)
    f3 = _mm(f, W1c_ref[...]) + b1c_ref[...]

    W2af = W2af_ref[...]
    o = jnp.maximum(_mm(E, _mm(code, W2ac_ref[...]))
                    + f3[:, 0:1] * W2af[0:1, :]
                    + f3[:, 1:2] * W2af[1:2, :]
                    + f3[:, 2:3] * W2af[2:3, :] + b2a_ref[...], 0.0)
    o = jnp.maximum(_mm(o, W2b_ref[...]) + b2b_ref[...], 0.0)
    o3 = _mm(o, W2c_ref[...]) + b2c_ref[...]
    out_ref[...] = o3 * scale + meanE


def _fold(xr, aux, weights):
    nprog = (_B * _NC) // _PB
    full = lambda shape: pl.BlockSpec(shape, lambda i: tuple(0 for _ in shape))
    in_specs = [
        pl.BlockSpec((_R, 3), lambda i: (i, 0)),
        pl.BlockSpec((_PB, 16), lambda i: (i, 0)),
        full(aux.shape),
        full((_NP, 2)),
    ] + [full(w.shape) for w in weights]
    return pl.pallas_call(
        _fold_body,
        grid=(nprog,),
        in_specs=in_specs,
        out_specs=pl.BlockSpec((_R, 3), lambda i: (i, 0)),
        out_shape=jax.ShapeDtypeStruct((_B * _NC * _NP, 3), jnp.float32),
    )(xr, aux, aux, jnp.asarray(_GRID), *weights)


def kernel(data, We1, be1, We2, be2, We3, be3, Wf1a, bf1a, Wf1b, bf1b, Wf1c,
           bf1c, Wf2a, bf2a, Wf2b, bf2b, Wf2c, bf2c):
    data3 = jnp.transpose(data, (1, 0, 2))
    cen = _fps(data3)
    perm = jax.random.permutation(jax.random.key(1), _NC)
    centers = jnp.transpose(cen, (1, 2, 0))[:, perm]
    d, t = _kthresh(data, centers)
    nq = _B * _NC
    t16 = jnp.broadcast_to(t.reshape(nq, 1), (nq, 16)).reshape(-1)
    outx, outy, outz, aux = _sc_select(d.reshape(nq, _N), t16,
                                       data.reshape(_B * 3, _N))
    xr = jnp.stack([outx, outy, outz], axis=-1)
    weights = [
        We1, be1.reshape(1, -1), We2, be2.reshape(1, -1), We3,
        be3.reshape(1, -1), Wf1a[:128], Wf1a[128:130], bf1a.reshape(1, -1),
        Wf1b, bf1b.reshape(1, -1), Wf1c, bf1c.reshape(1, -1), Wf2a[:128],
        Wf2a[128:131], bf2a.reshape(1, -1), Wf2b, bf2b.reshape(1, -1), Wf2c,
        bf2c.reshape(1, -1),
    ]
    out = _fold(xr, aux.reshape(nq, 16), weights)
    return out.reshape(_B, _NC, _NP, 3)


# speedup vs baseline: 1.1195x; 1.1195x over previous
import functools
import numpy as np
import jax
import jax.numpy as jnp
from jax import lax
from jax.experimental import pallas as pl
from jax.experimental.pallas import tpu as pltpu
from jax.experimental.pallas import tpu_sc as plsc

_B, _N, _NC, _NP = 4, 8192, 256, 64
_CB = 64
_PB = 64
_R = _PB * _NP
_NW = 32
_QPW = (_B * _NC) // _NW
_GW = 96
_BISECT = 20


def _grid_np(n_p):
    side = int(np.sqrt(n_p))
    xs = np.linspace(-1.0, 1.0, side, dtype=np.float32)
    gx, gy = np.meshgrid(xs, xs, indexing='ij')
    return np.stack([gx.reshape(-1), gy.reshape(-1)], axis=-1)


_GRID = _grid_np(_NP)


def _fps_body(pts_ref, cen_ref):
    px = pts_ref[0]
    py = pts_ref[1]
    pz = pts_ref[2]
    iota = lax.broadcasted_iota(jnp.int32, (_B, _N), 1)
    iota_c = lax.broadcasted_iota(jnp.int32, (_B, _NC), 1)
    d = (px - px[:, :1]) ** 2 + (py - py[:, :1]) ** 2 + (pz - pz[:, :1]) ** 2
    cenx = jnp.where(iota_c == 0, px[:, :1], 0.0)
    ceny = jnp.where(iota_c == 0, py[:, :1], 0.0)
    cenz = jnp.where(iota_c == 0, pz[:, :1], 0.0)

    def step(t, carry):
        d, cenx, ceny, cenz = carry
        m = jnp.max(d, axis=1, keepdims=True)
        far = jnp.min(jnp.where(d == m, iota, jnp.int32(_N)), axis=1,
                      keepdims=True)
        oh = iota == far
        cx = jnp.sum(jnp.where(oh, px, 0.0), axis=1, keepdims=True)
        cy = jnp.sum(jnp.where(oh, py, 0.0), axis=1, keepdims=True)
        cz = jnp.sum(jnp.where(oh, pz, 0.0), axis=1, keepdims=True)
        dn = (px - cx) ** 2 + (py - cy) ** 2 + (pz - cz) ** 2
        d = jnp.minimum(d, dn)
        sel = iota_c == t
        cenx = jnp.where(sel, cx, cenx)
        ceny = jnp.where(sel, cy, ceny)
        cenz = jnp.where(sel, cz, cenz)
        return d, cenx, ceny, cenz

    d, cenx, ceny, cenz = lax.fori_loop(1, _NC, step, (d, cenx, ceny, cenz))
    cen_ref[0] = cenx
    cen_ref[1] = ceny
    cen_ref[2] = cenz


def _fps(data3):
    return pl.pallas_call(
        _fps_body,
        out_shape=jax.ShapeDtypeStruct((3, _B, _NC), jnp.float32),
    )(data3)


def _kthresh_body(pts_ref, cen_ref, d_ref, t_ref):
    px = pts_ref[0, 0:1, :]
    py = pts_ref[0, 1:2, :]
    pz = pts_ref[0, 2:3, :]
    p3 = pts_ref[0]
    q = cen_ref[0]
    qsq = jnp.sum(q * q, axis=1, keepdims=True)
    psq = px * px + py * py + pz * pz
    qp = lax.dot_general(q, p3, (((1,), (0,)), ((), ())))
    d = qsq + psq - 2.0 * qp
    d_ref[0] = d

    hi = jnp.max(jnp.min(d.reshape(_CB, _NP, _N // _NP), axis=2), axis=1,
                 keepdims=True)
    lo = jnp.zeros_like(hi)

    def it(_, lohi):
        lo, hi = lohi
        mid = 0.5 * (lo + hi)
        cnt = jnp.sum(jnp.where(d <= mid, 1.0, 0.0), axis=1, keepdims=True)
        ge = cnt >= jnp.float32(_NP)
        hi = jnp.where(ge, mid, hi)
        lo = jnp.where(ge, lo, mid)
        return lo, hi

    lo, hi = lax.fori_loop(0, _BISECT, it, (lo, hi))
    t_ref[0] = hi


def _kthresh(data, centers):
    nblk = _NC // _CB
    return pl.pallas_call(
        _kthresh_body,
        grid=(_B, nblk),
        in_specs=[
            pl.BlockSpec((1, 3, _N), lambda b, c: (b, 0, 0)),
            pl.BlockSpec((1, _CB, 3), lambda b, c: (b, c, 0)),
        ],
        out_specs=[
            pl.BlockSpec((1, _CB, _N), lambda b, c: (b, c, 0)),
            pl.BlockSpec((1, _CB, 1), lambda b, c: (b * nblk + c, 0, 0)),
        ],
        out_shape=[
            jax.ShapeDtypeStruct((_B, _NC, _N), jnp.float32),
            jax.ShapeDtypeStruct((_B * nblk, _CB, 1), jnp.float32),
        ],
    )(data, centers)


def _sc_body(d_hbm, t16_hbm, data_hbm, outx_hbm, outy_hbm, outz_hbm, aux_hbm,
             pxb, pyb, pzb, dbuf, tbuf, gxb, gyb, gzb, auxb):
    cid = lax.axis_index("c")
    sid = lax.axis_index("s")
    wid = sid * 2 + cid
    b = wid // (_NW // _B)
    pltpu.sync_copy(data_hbm.at[b * 3], pxb)
    pltpu.sync_copy(data_hbm.at[b * 3 + 1], pyb)
    pltpu.sync_copy(data_hbm.at[b * 3 + 2], pzb)
    woff = pl.multiple_of(wid * (_QPW * 16), 512)
    pltpu.sync_copy(t16_hbm.at[pl.ds(woff, _QPW * 16)], tbuf)
    iota16 = lax.iota(jnp.int32, 16)

    def per_query(i, carry):
        q = wid * _QPW + i
        pltpu.sync_copy(d_hbm.at[q], dbuf)
        tv = tbuf[pl.ds(i * 16, 16)]
        base = i * _GW

        lim = lax.broadcast(base + (_GW - 1), (16,))

        one16 = jnp.ones((16,), jnp.int32)

        def scan_vreg(j, cur):
            v = dbuf[pl.ds(j * 16, 16)]
            m = v <= tv
            ranks = plsc.cumsum(m.astype(jnp.int32))
            pos = jnp.minimum(cur + ranks - one16, lim)
            gx = pxb[pl.ds(j * 16, 16)]
            gy = pyb[pl.ds(j * 16, 16)]
            gz = pzb[pl.ds(j * 16, 16)]
            plsc.store_scatter(gxb, [pos], gx, mask=m)
            plsc.store_scatter(gyb, [pos], gy, mask=m)
            plsc.store_scatter(gzb, [pos], gz, mask=m)
            return cur + plsc.all_reduce_population_count(m)

        cur0 = lax.broadcast(base, (16,))
        plsc.parallel_loop(0, _N // 16, carry=cur0, unroll=8)(scan_vreg)

        bofs = pl.multiple_of(base, 8)
        x0 = gxb[pl.ds(bofs, 16)]
        x1 = gxb[pl.ds(bofs + 16, 16)]
        x2 = gxb[pl.ds(bofs + 32, 16)]
        x3 = gxb[pl.ds(bofs + 48, 16)]
        y0 = gyb[pl.ds(bofs, 16)]
        y1 = gyb[pl.ds(bofs + 16, 16)]
        y2 = gyb[pl.ds(bofs + 32, 16)]
        y3 = gyb[pl.ds(bofs + 48, 16)]
        z0 = gzb[pl.ds(bofs, 16)]
        z1 = gzb[pl.ds(bofs + 16, 16)]
        z2 = gzb[pl.ds(bofs + 32, 16)]
        z3 = gzb[pl.ds(bofs + 48, 16)]
        mxv = lax.broadcast(jnp.sum((x0 + x1) + (x2 + x3)) * (1.0 / _NP), (16,))
        myv = lax.broadcast(jnp.sum((y0 + y1) + (y2 + y3)) * (1.0 / _NP), (16,))
        mzv = lax.broadcast(jnp.sum((z0 + z1) + (z2 + z3)) * (1.0 / _NP), (16,))

        def _nsq(a, b, c):
            da, db, dc = a - mxv, b - myv, c - mzv
            return da * da + db * db + dc * dc

        n0 = _nsq(x0, y0, z0)
        n1 = _nsq(x1, y1, z1)
        n2 = _nsq(x2, y2, z2)
        n3 = _nsq(x3, y3, z3)
        msqv = lax.broadcast(
            jnp.max(jnp.maximum(jnp.maximum(n0, n1), jnp.maximum(n2, n3))),
            (16,))
        zv = jnp.zeros((16,), jnp.float32)
        av = jnp.where(iota16 == 0, mxv,
                       jnp.where(iota16 == 1, myv,
                                 jnp.where(iota16 == 2, mzv,
                                           jnp.where(iota16 == 3, msqv, zv))))
        auxb[pl.ds(i * 16, 16)] = av

        oofs = pl.multiple_of(q * _NP, 64)
        pltpu.sync_copy(gxb.at[pl.ds(bofs, _NP)], outx_hbm.at[pl.ds(oofs, _NP)])
        pltpu.sync_copy(gyb.at[pl.ds(bofs, _NP)], outy_hbm.at[pl.ds(oofs, _NP)])
        pltpu.sync_copy(gzb.at[pl.ds(bofs, _NP)], outz_hbm.at[pl.ds(oofs, _NP)])
        return carry

    lax.fori_loop(0, _QPW, per_query, 0)
    pltpu.sync_copy(auxb, aux_hbm.at[pl.ds(woff, _QPW * 16)])


def _sc_select(d_flat, t16, data):
    nq = _B * _NC
    f = pl.kernel(
        _sc_body,
        mesh=plsc.VectorSubcoreMesh(core_axis_name="c", subcore_axis_name="s"),
        compiler_params=pltpu.CompilerParams(needs_layout_passes=False),
        out_type=[
            jax.ShapeDtypeStruct((nq * _NP,), jnp.float32),
            jax.ShapeDtypeStruct((nq * _NP,), jnp.float32),
            jax.ShapeDtypeStruct((nq * _NP,), jnp.float32),
            jax.ShapeDtypeStruct((nq * 16,), jnp.float32),
        ],
        scratch_types=[
            pltpu.VMEM((_N,), jnp.float32),
            pltpu.VMEM((_N,), jnp.float32),
            pltpu.VMEM((_N,), jnp.float32),
            pltpu.VMEM((_N,), jnp.float32),
            pltpu.VMEM((_QPW * 16,), jnp.float32),
            pltpu.VMEM((_QPW * _GW,), jnp.float32),
            pltpu.VMEM((_QPW * _GW,), jnp.float32),
            pltpu.VMEM((_QPW * _GW,), jnp.float32),
            pltpu.VMEM((_QPW * 16,), jnp.float32),
        ],
    )
    return f(d_flat, t16, data)


def _mm(a, b):
    return lax.dot_general(a, b, (((1,), (0,)), ((), ())),
                           preferred_element_type=jnp.float32)


def _fold_body(xr_ref, auxb_ref, auxf_ref, grid_ref,
               We1_ref, be1_ref, We2_ref, be2_ref, We3_ref, be3_ref,
               W1ac_ref, W1ag_ref, b1a_ref, W1b_ref, b1b_ref, W1c_ref,
               b1c_ref, W2ac_ref, W2af_ref, b2a_ref, W2b_ref, b2b_ref,
               W2c_ref, b2c_ref, out_ref):
    scale = jnp.sqrt(jnp.max(auxf_ref[:, 3:4]))
    inv = 1.0 / scale

    riota = lax.broadcasted_iota(jnp.int32, (_R, _PB), 0)
    piota = lax.broadcasted_iota(jnp.int32, (_R, _PB), 1)
    E = (riota // _NP == piota).astype(jnp.float32)
    riota2 = lax.broadcasted_iota(jnp.int32, (_R, _NP), 0)
    kiota = lax.broadcasted_iota(jnp.int32, (_R, _NP), 1)
    S = (riota2 % _NP == kiota).astype(jnp.float32)

    meanE = _mm(E, auxb_ref[:, 0:3])
    xx = (xr_ref[:, 0:1] - meanE[:, 0:1]) * inv
    yy = (xr_ref[:, 1:2] - meanE[:, 1:2]) * inv
    zz = (xr_ref[:, 2:3] - meanE[:, 2:3]) * inv
    We1 = We1_ref[...]
    h = jnp.maximum(xx * We1[0:1, :] + yy * We1[1:2, :] + zz * We1[2:3, :]
                    + be1_ref[...], 0.0)
    h = jnp.maximum(_mm(h, We2_ref[...]) + be2_ref[...], 0.0)
    h = _mm(h, We3_ref[...]) + be3_ref[...]
    code = jnp.max(h.reshape(_PB, _NP, 128), axis=1)

    g = grid_ref[...]
    W1ag = W1ag_ref[...]
    gW = g[:, 0:1] * W1ag[0:1, :] + g[:, 1:2] * W1ag[1:2, :]
    f = jnp.maximum(_mm(E, _mm(code, W1ac_ref[...])) + _mm(S, gW)
                    + b1a_ref[...], 0.0)
    f = jnp.maximum(_mm(f, W1b_ref[...]) + b1b_ref[...], 0.0)
    f3 = _mm(f, W1c_ref[...]) + b1c_ref[...]

    W2af = W2af_ref[...]
    o = jnp.maximum(_mm(E, _mm(code, W2ac_ref[...]))
                    + f3[:, 0:1] * W2af[0:1, :]
                    + f3[:, 1:2] * W2af[1:2, :]
                    + f3[:, 2:3] * W2af[2:3, :] + b2a_ref[...], 0.0)
    o = jnp.maximum(_mm(o, W2b_ref[...]) + b2b_ref[...], 0.0)
    o3 = _mm(o, W2c_ref[...]) + b2c_ref[...]
    out_ref[...] = o3 * scale + meanE


def _fold(xr, aux, weights):
    nprog = (_B * _NC) // _PB
    full = lambda shape: pl.BlockSpec(shape, lambda i: tuple(0 for _ in shape))
    in_specs = [
        pl.BlockSpec((_R, 3), lambda i: (i, 0)),
        pl.BlockSpec((_PB, 16), lambda i: (i, 0)),
        full(aux.shape),
        full((_NP, 2)),
    ] + [full(w.shape) for w in weights]
    return pl.pallas_call(
        _fold_body,
        grid=(nprog,),
        in_specs=in_specs,
        out_specs=pl.BlockSpec((_R, 3), lambda i: (i, 0)),
        out_shape=jax.ShapeDtypeStruct((_B * _NC * _NP, 3), jnp.float32),
    )(xr, aux, aux, jnp.asarray(_GRID), *weights)


def kernel(data, We1, be1, We2, be2, We3, be3, Wf1a, bf1a, Wf1b, bf1b, Wf1c,
           bf1c, Wf2a, bf2a, Wf2b, bf2b, Wf2c, bf2c):
    data3 = jnp.transpose(data, (1, 0, 2))
    cen = _fps(data3)
    perm = jax.random.permutation(jax.random.key(1), _NC)
    centers = jnp.transpose(cen, (1, 2, 0))[:, perm]
    d, t = _kthresh(data, centers)
    nq = _B * _NC
    t16 = jnp.broadcast_to(t.reshape(nq, 1), (nq, 16)).reshape(-1)
    outx, outy, outz, aux = _sc_select(d.reshape(nq, _N), t16,
                                       data.reshape(_B * 3, _N))
    xr = jnp.stack([outx, outy, outz], axis=-1)
    weights = [
        We1, be1.reshape(1, -1), We2, be2.reshape(1, -1), We3,
        be3.reshape(1, -1), Wf1a[:128], Wf1a[128:130], bf1a.reshape(1, -1),
        Wf1b, bf1b.reshape(1, -1), Wf1c, bf1c.reshape(1, -1), Wf2a[:128],
        Wf2a[128:131], bf2a.reshape(1, -1), Wf2b, bf2b.reshape(1, -1), Wf2c,
        bf2c.reshape(1, -1),
    ]
    out = _fold(xr, aux.reshape(nq, 16), weights)
    return out.reshape(_B, _NC, _NP, 3)


# speedup vs baseline: 1.1223x; 1.0025x over previous
import functools
import numpy as np
import jax
import jax.numpy as jnp
from jax import lax
from jax.experimental import pallas as pl
from jax.experimental.pallas import tpu as pltpu
from jax.experimental.pallas import tpu_sc as plsc

_B, _N, _NC, _NP = 4, 8192, 256, 64
_CB = 64
_PB = 128
_R = _PB * _NP
_NW = 32
_QPW = (_B * _NC) // _NW
_GW = 96
_BISECT = 20


def _grid_np(n_p):
    side = int(np.sqrt(n_p))
    xs = np.linspace(-1.0, 1.0, side, dtype=np.float32)
    gx, gy = np.meshgrid(xs, xs, indexing='ij')
    return np.stack([gx.reshape(-1), gy.reshape(-1)], axis=-1)


_GRID = _grid_np(_NP)


def _fps_body(pts_ref, cen_ref):
    px = pts_ref[0]
    py = pts_ref[1]
    pz = pts_ref[2]
    iota = lax.broadcasted_iota(jnp.int32, (_B, _N), 1)
    iota_c = lax.broadcasted_iota(jnp.int32, (_B, _NC), 1)
    d = (px - px[:, :1]) ** 2 + (py - py[:, :1]) ** 2 + (pz - pz[:, :1]) ** 2
    cenx = jnp.where(iota_c == 0, px[:, :1], 0.0)
    ceny = jnp.where(iota_c == 0, py[:, :1], 0.0)
    cenz = jnp.where(iota_c == 0, pz[:, :1], 0.0)

    def step(t, carry):
        d, cenx, ceny, cenz = carry
        m = jnp.max(d, axis=1, keepdims=True)
        far = jnp.min(jnp.where(d == m, iota, jnp.int32(_N)), axis=1,
                      keepdims=True)
        oh = iota == far
        cx = jnp.sum(jnp.where(oh, px, 0.0), axis=1, keepdims=True)
        cy = jnp.sum(jnp.where(oh, py, 0.0), axis=1, keepdims=True)
        cz = jnp.sum(jnp.where(oh, pz, 0.0), axis=1, keepdims=True)
        dn = (px - cx) ** 2 + (py - cy) ** 2 + (pz - cz) ** 2
        d = jnp.minimum(d, dn)
        sel = iota_c == t
        cenx = jnp.where(sel, cx, cenx)
        ceny = jnp.where(sel, cy, ceny)
        cenz = jnp.where(sel, cz, cenz)
        return d, cenx, ceny, cenz

    d, cenx, ceny, cenz = lax.fori_loop(1, _NC, step, (d, cenx, ceny, cenz))
    cen_ref[0] = cenx
    cen_ref[1] = ceny
    cen_ref[2] = cenz


def _fps(data3):
    return pl.pallas_call(
        _fps_body,
        out_shape=jax.ShapeDtypeStruct((3, _B, _NC), jnp.float32),
    )(data3)


def _kthresh_body(pts_ref, cen_ref, d_ref, t_ref):
    px = pts_ref[0, 0:1, :]
    py = pts_ref[0, 1:2, :]
    pz = pts_ref[0, 2:3, :]
    p3 = pts_ref[0]
    q = cen_ref[0]
    qsq = jnp.sum(q * q, axis=1, keepdims=True)
    psq = px * px + py * py + pz * pz
    qp = lax.dot_general(q, p3, (((1,), (0,)), ((), ())))
    d = qsq + psq - 2.0 * qp
    d_ref[0] = d

    hi = jnp.max(jnp.min(d.reshape(_CB, _NP, _N // _NP), axis=2), axis=1,
                 keepdims=True)
    lo = jnp.zeros_like(hi)

    def it(_, lohi):
        lo, hi = lohi
        mid = 0.5 * (lo + hi)
        cnt = jnp.sum(jnp.where(d <= mid, 1.0, 0.0), axis=1, keepdims=True)
        ge = cnt >= jnp.float32(_NP)
        hi = jnp.where(ge, mid, hi)
        lo = jnp.where(ge, lo, mid)
        return lo, hi

    lo, hi = lax.fori_loop(0, _BISECT, it, (lo, hi))
    t_ref[0] = hi


def _kthresh(data, centers):
    nblk = _NC // _CB
    return pl.pallas_call(
        _kthresh_body,
        grid=(_B, nblk),
        in_specs=[
            pl.BlockSpec((1, 3, _N), lambda b, c: (b, 0, 0)),
            pl.BlockSpec((1, _CB, 3), lambda b, c: (b, c, 0)),
        ],
        out_specs=[
            pl.BlockSpec((1, _CB, _N), lambda b, c: (b, c, 0)),
            pl.BlockSpec((1, _CB, 1), lambda b, c: (b * nblk + c, 0, 0)),
        ],
        out_shape=[
            jax.ShapeDtypeStruct((_B, _NC, _N), jnp.float32),
            jax.ShapeDtypeStruct((_B * nblk, _CB, 1), jnp.float32),
        ],
    )(data, centers)


def _sc_body(d_hbm, t16_hbm, data_hbm, outx_hbm, outy_hbm, outz_hbm, aux_hbm,
             pxb, pyb, pzb, dbuf, tbuf, gxb, gyb, gzb, auxb):
    cid = lax.axis_index("c")
    sid = lax.axis_index("s")
    wid = sid * 2 + cid
    b = wid // (_NW // _B)
    pltpu.sync_copy(data_hbm.at[b * 3], pxb)
    pltpu.sync_copy(data_hbm.at[b * 3 + 1], pyb)
    pltpu.sync_copy(data_hbm.at[b * 3 + 2], pzb)
    woff = pl.multiple_of(wid * (_QPW * 16), 512)
    pltpu.sync_copy(t16_hbm.at[pl.ds(woff, _QPW * 16)], tbuf)
    iota16 = lax.iota(jnp.int32, 16)

    def per_query(i, carry):
        q = wid * _QPW + i
        pltpu.sync_copy(d_hbm.at[q], dbuf)
        tv = tbuf[pl.ds(i * 16, 16)]
        base = i * _GW

        lim = lax.broadcast(base + (_GW - 1), (16,))

        one16 = jnp.ones((16,), jnp.int32)

        def scan_vreg(j, cur):
            v = dbuf[pl.ds(j * 16, 16)]
            m = v <= tv
            ranks = plsc.cumsum(m.astype(jnp.int32))
            pos = jnp.minimum(cur + ranks - one16, lim)
            gx = pxb[pl.ds(j * 16, 16)]
            gy = pyb[pl.ds(j * 16, 16)]
            gz = pzb[pl.ds(j * 16, 16)]
            plsc.store_scatter(gxb, [pos], gx, mask=m)
            plsc.store_scatter(gyb, [pos], gy, mask=m)
            plsc.store_scatter(gzb, [pos], gz, mask=m)
            return cur + plsc.all_reduce_population_count(m)

        cur0 = lax.broadcast(base, (16,))
        plsc.parallel_loop(0, _N // 16, carry=cur0, unroll=8)(scan_vreg)

        bofs = pl.multiple_of(base, 8)
        x0 = gxb[pl.ds(bofs, 16)]
        x1 = gxb[pl.ds(bofs + 16, 16)]
        x2 = gxb[pl.ds(bofs + 32, 16)]
        x3 = gxb[pl.ds(bofs + 48, 16)]
        y0 = gyb[pl.ds(bofs, 16)]
        y1 = gyb[pl.ds(bofs + 16, 16)]
        y2 = gyb[pl.ds(bofs + 32, 16)]
        y3 = gyb[pl.ds(bofs + 48, 16)]
        z0 = gzb[pl.ds(bofs, 16)]
        z1 = gzb[pl.ds(bofs + 16, 16)]
        z2 = gzb[pl.ds(bofs + 32, 16)]
        z3 = gzb[pl.ds(bofs + 48, 16)]
        mxv = lax.broadcast(jnp.sum((x0 + x1) + (x2 + x3)) * (1.0 / _NP), (16,))
        myv = lax.broadcast(jnp.sum((y0 + y1) + (y2 + y3)) * (1.0 / _NP), (16,))
        mzv = lax.broadcast(jnp.sum((z0 + z1) + (z2 + z3)) * (1.0 / _NP), (16,))

        def _nsq(a, b, c):
            da, db, dc = a - mxv, b - myv, c - mzv
            return da * da + db * db + dc * dc

        n0 = _nsq(x0, y0, z0)
        n1 = _nsq(x1, y1, z1)
        n2 = _nsq(x2, y2, z2)
        n3 = _nsq(x3, y3, z3)
        msqv = lax.broadcast(
            jnp.max(jnp.maximum(jnp.maximum(n0, n1), jnp.maximum(n2, n3))),
            (16,))
        zv = jnp.zeros((16,), jnp.float32)
        av = jnp.where(iota16 == 0, mxv,
                       jnp.where(iota16 == 1, myv,
                                 jnp.where(iota16 == 2, mzv,
                                           jnp.where(iota16 == 3, msqv, zv))))
        auxb[pl.ds(i * 16, 16)] = av

        oofs = pl.multiple_of(q * _NP, 64)
        pltpu.sync_copy(gxb.at[pl.ds(bofs, _NP)], outx_hbm.at[pl.ds(oofs, _NP)])
        pltpu.sync_copy(gyb.at[pl.ds(bofs, _NP)], outy_hbm.at[pl.ds(oofs, _NP)])
        pltpu.sync_copy(gzb.at[pl.ds(bofs, _NP)], outz_hbm.at[pl.ds(oofs, _NP)])
        return carry

    lax.fori_loop(0, _QPW, per_query, 0)
    pltpu.sync_copy(auxb, aux_hbm.at[pl.ds(woff, _QPW * 16)])


def _sc_select(d_flat, t16, data):
    nq = _B * _NC
    f = pl.kernel(
        _sc_body,
        mesh=plsc.VectorSubcoreMesh(core_axis_name="c", subcore_axis_name="s"),
        compiler_params=pltpu.CompilerParams(needs_layout_passes=False),
        out_type=[
            jax.ShapeDtypeStruct((nq * _NP,), jnp.float32),
            jax.ShapeDtypeStruct((nq * _NP,), jnp.float32),
            jax.ShapeDtypeStruct((nq * _NP,), jnp.float32),
            jax.ShapeDtypeStruct((nq * 16,), jnp.float32),
        ],
        scratch_types=[
            pltpu.VMEM((_N,), jnp.float32),
            pltpu.VMEM((_N,), jnp.float32),
            pltpu.VMEM((_N,), jnp.float32),
            pltpu.VMEM((_N,), jnp.float32),
            pltpu.VMEM((_QPW * 16,), jnp.float32),
            pltpu.VMEM((_QPW * _GW,), jnp.float32),
            pltpu.VMEM((_QPW * _GW,), jnp.float32),
            pltpu.VMEM((_QPW * _GW,), jnp.float32),
            pltpu.VMEM((_QPW * 16,), jnp.float32),
        ],
    )
    return f(d_flat, t16, data)


def _mm(a, b):
    return lax.dot_general(a, b, (((1,), (0,)), ((), ())),
                           preferred_element_type=jnp.float32)


def _fold_body(xr_ref, auxb_ref, auxf_ref, grid_ref,
               We1_ref, be1_ref, We2_ref, be2_ref, We3_ref, be3_ref,
               W1ac_ref, W1ag_ref, b1a_ref, W1b_ref, b1b_ref, W1c_ref,
               b1c_ref, W2ac_ref, W2af_ref, b2a_ref, W2b_ref, b2b_ref,
               W2c_ref, b2c_ref, out_ref):
    scale = jnp.sqrt(jnp.max(auxf_ref[:, 3:4]))
    inv = 1.0 / scale

    riota = lax.broadcasted_iota(jnp.int32, (_R, _PB), 0)
    piota = lax.broadcasted_iota(jnp.int32, (_R, _PB), 1)
    E = (riota // _NP == piota).astype(jnp.float32)
    riota2 = lax.broadcasted_iota(jnp.int32, (_R, _NP), 0)
    kiota = lax.broadcasted_iota(jnp.int32, (_R, _NP), 1)
    S = (riota2 % _NP == kiota).astype(jnp.float32)

    meanE = _mm(E, auxb_ref[:, 0:3])
    xx = (xr_ref[:, 0:1] - meanE[:, 0:1]) * inv
    yy = (xr_ref[:, 1:2] - meanE[:, 1:2]) * inv
    zz = (xr_ref[:, 2:3] - meanE[:, 2:3]) * inv
    We1 = We1_ref[...]
    h = jnp.maximum(xx * We1[0:1, :] + yy * We1[1:2, :] + zz * We1[2:3, :]
                    + be1_ref[...], 0.0)
    h = jnp.maximum(_mm(h, We2_ref[...]) + be2_ref[...], 0.0)
    h = _mm(h, We3_ref[...]) + be3_ref[...]
    code = jnp.max(h.reshape(_PB, _NP, 128), axis=1)

    g = grid_ref[...]
    W1ag = W1ag_ref[...]
    gW = g[:, 0:1] * W1ag[0:1, :] + g[:, 1:2] * W1ag[1:2, :]
    f = jnp.maximum(_mm(E, _mm(code, W1ac_ref[...])) + _mm(S, gW)
                    + b1a_ref[...], 0.0)
    f = jnp.maximum(_mm(f, W1b_ref[...]) + b1b_ref[...], 0.0)
    f3 = _mm(f, W1c_ref[...]) + b1c_ref[...]

    W2af = W2af_ref[...]
    o = jnp.maximum(_mm(E, _mm(code, W2ac_ref[...]))
                    + f3[:, 0:1] * W2af[0:1, :]
                    + f3[:, 1:2] * W2af[1:2, :]
                    + f3[:, 2:3] * W2af[2:3, :] + b2a_ref[...], 0.0)
    o = jnp.maximum(_mm(o, W2b_ref[...]) + b2b_ref[...], 0.0)
    o3 = _mm(o, W2c_ref[...]) + b2c_ref[...]
    out_ref[...] = o3 * scale + meanE


def _fold(xr, aux, weights):
    nprog = (_B * _NC) // _PB
    full = lambda shape: pl.BlockSpec(shape, lambda i: tuple(0 for _ in shape))
    in_specs = [
        pl.BlockSpec((_R, 3), lambda i: (i, 0)),
        pl.BlockSpec((_PB, 16), lambda i: (i, 0)),
        full(aux.shape),
        full((_NP, 2)),
    ] + [full(w.shape) for w in weights]
    return pl.pallas_call(
        _fold_body,
        grid=(nprog,),
        in_specs=in_specs,
        out_specs=pl.BlockSpec((_R, 3), lambda i: (i, 0)),
        out_shape=jax.ShapeDtypeStruct((_B * _NC * _NP, 3), jnp.float32),
    )(xr, aux, aux, jnp.asarray(_GRID), *weights)


def kernel(data, We1, be1, We2, be2, We3, be3, Wf1a, bf1a, Wf1b, bf1b, Wf1c,
           bf1c, Wf2a, bf2a, Wf2b, bf2b, Wf2c, bf2c):
    data3 = jnp.transpose(data, (1, 0, 2))
    cen = _fps(data3)
    perm = jax.random.permutation(jax.random.key(1), _NC)
    centers = jnp.transpose(cen, (1, 2, 0))[:, perm]
    d, t = _kthresh(data, centers)
    nq = _B * _NC
    t16 = jnp.broadcast_to(t.reshape(nq, 1), (nq, 16)).reshape(-1)
    outx, outy, outz, aux = _sc_select(d.reshape(nq, _N), t16,
                                       data.reshape(_B * 3, _N))
    xr = jnp.stack([outx, outy, outz], axis=-1)
    weights = [
        We1, be1.reshape(1, -1), We2, be2.reshape(1, -1), We3,
        be3.reshape(1, -1), Wf1a[:128], Wf1a[128:130], bf1a.reshape(1, -1),
        Wf1b, bf1b.reshape(1, -1), Wf1c, bf1c.reshape(1, -1), Wf2a[:128],
        Wf2a[128:131], bf2a.reshape(1, -1), Wf2b, bf2b.reshape(1, -1), Wf2c,
        bf2c.reshape(1, -1),
    ]
    out = _fold(xr, aux.reshape(nq, 16), weights)
    return out.reshape(_B, _NC, _NP, 3)


# speedup vs baseline: 1.1696x; 1.0421x over previous
import functools
import numpy as np
import jax
import jax.numpy as jnp
from jax import lax
from jax.experimental import pallas as pl
from jax.experimental.pallas import tpu as pltpu
from jax.experimental.pallas import tpu_sc as plsc

_B, _N, _NC, _NP = 4, 8192, 256, 64
_CB = 64
_PB = 128
_R = _PB * _NP
_NW = 32
_QPW = (_B * _NC) // _NW
_GW = 96
_BISECT = 16


def _grid_np(n_p):
    side = int(np.sqrt(n_p))
    xs = np.linspace(-1.0, 1.0, side, dtype=np.float32)
    gx, gy = np.meshgrid(xs, xs, indexing='ij')
    return np.stack([gx.reshape(-1), gy.reshape(-1)], axis=-1)


_GRID = _grid_np(_NP)


def _fps_body(pts_ref, cen_ref):
    px = pts_ref[0]
    py = pts_ref[1]
    pz = pts_ref[2]
    iota = lax.broadcasted_iota(jnp.int32, (_B, _N), 1)
    iota_c = lax.broadcasted_iota(jnp.int32, (_B, _NC), 1)
    d = (px - px[:, :1]) ** 2 + (py - py[:, :1]) ** 2 + (pz - pz[:, :1]) ** 2
    cenx = jnp.where(iota_c == 0, px[:, :1], 0.0)
    ceny = jnp.where(iota_c == 0, py[:, :1], 0.0)
    cenz = jnp.where(iota_c == 0, pz[:, :1], 0.0)

    def step(t, carry):
        d, cenx, ceny, cenz = carry
        m = jnp.max(d, axis=1, keepdims=True)
        far = jnp.min(jnp.where(d == m, iota, jnp.int32(_N)), axis=1,
                      keepdims=True)
        oh = iota == far
        cx = jnp.sum(jnp.where(oh, px, 0.0), axis=1, keepdims=True)
        cy = jnp.sum(jnp.where(oh, py, 0.0), axis=1, keepdims=True)
        cz = jnp.sum(jnp.where(oh, pz, 0.0), axis=1, keepdims=True)
        dn = (px - cx) ** 2 + (py - cy) ** 2 + (pz - cz) ** 2
        d = jnp.minimum(d, dn)
        sel = iota_c == t
        cenx = jnp.where(sel, cx, cenx)
        ceny = jnp.where(sel, cy, ceny)
        cenz = jnp.where(sel, cz, cenz)
        return d, cenx, ceny, cenz

    d, cenx, ceny, cenz = lax.fori_loop(1, _NC, step, (d, cenx, ceny, cenz))
    cen_ref[0] = cenx
    cen_ref[1] = ceny
    cen_ref[2] = cenz


def _fps(data3):
    return pl.pallas_call(
        _fps_body,
        out_shape=jax.ShapeDtypeStruct((3, _B, _NC), jnp.float32),
    )(data3)


def _kthresh_body(pts_ref, cen_ref, d_ref, t_ref):
    px = pts_ref[0, 0:1, :]
    py = pts_ref[0, 1:2, :]
    pz = pts_ref[0, 2:3, :]
    p3 = pts_ref[0]
    q = cen_ref[0]
    qsq = jnp.sum(q * q, axis=1, keepdims=True)
    psq = px * px + py * py + pz * pz
    qp = lax.dot_general(q, p3, (((1,), (0,)), ((), ())))
    d = qsq + psq - 2.0 * qp
    d_ref[0] = d

    hi = jnp.max(jnp.min(d.reshape(_CB, _NP, _N // _NP), axis=2), axis=1,
                 keepdims=True)
    lo = jnp.zeros_like(hi)

    def it(_, lohi):
        lo, hi = lohi
        mid = 0.5 * (lo + hi)
        cnt = jnp.sum(jnp.where(d <= mid, 1.0, 0.0), axis=1, keepdims=True)
        ge = cnt >= jnp.float32(_NP)
        hi = jnp.where(ge, mid, hi)
        lo = jnp.where(ge, lo, mid)
        return lo, hi

    lo, hi = lax.fori_loop(0, _BISECT, it, (lo, hi))
    t_ref[0] = hi


def _kthresh(data, centers):
    nblk = _NC // _CB
    return pl.pallas_call(
        _kthresh_body,
        grid=(_B, nblk),
        in_specs=[
            pl.BlockSpec((1, 3, _N), lambda b, c: (b, 0, 0)),
            pl.BlockSpec((1, _CB, 3), lambda b, c: (b, c, 0)),
        ],
        out_specs=[
            pl.BlockSpec((1, _CB, _N), lambda b, c: (b, c, 0)),
            pl.BlockSpec((1, _CB, 1), lambda b, c: (b * nblk + c, 0, 0)),
        ],
        out_shape=[
            jax.ShapeDtypeStruct((_B, _NC, _N), jnp.float32),
            jax.ShapeDtypeStruct((_B * nblk, _CB, 1), jnp.float32),
        ],
    )(data, centers)


def _sc_body(d_hbm, t16_hbm, data_hbm, outx_hbm, outy_hbm, outz_hbm, aux_hbm,
             pxb, pyb, pzb, dbuf, tbuf, gxb, gyb, gzb, auxb):
    cid = lax.axis_index("c")
    sid = lax.axis_index("s")
    wid = sid * 2 + cid
    b = wid // (_NW // _B)
    pltpu.sync_copy(data_hbm.at[b * 3], pxb)
    pltpu.sync_copy(data_hbm.at[b * 3 + 1], pyb)
    pltpu.sync_copy(data_hbm.at[b * 3 + 2], pzb)
    woff = pl.multiple_of(wid * (_QPW * 16), 512)
    pltpu.sync_copy(t16_hbm.at[pl.ds(woff, _QPW * 16)], tbuf)
    iota16 = lax.iota(jnp.int32, 16)

    def per_query(i, carry):
        q = wid * _QPW + i
        pltpu.sync_copy(d_hbm.at[q], dbuf)
        tv = tbuf[pl.ds(i * 16, 16)]
        base = i * _GW

        lim = lax.broadcast(base + (_GW - 1), (16,))

        one16 = jnp.ones((16,), jnp.int32)

        def scan_vreg(j, cur):
            v = dbuf[pl.ds(j * 16, 16)]
            m = v <= tv
            ranks = plsc.cumsum(m.astype(jnp.int32))
            pos = jnp.minimum(cur + ranks - one16, lim)
            gx = pxb[pl.ds(j * 16, 16)]
            gy = pyb[pl.ds(j * 16, 16)]
            gz = pzb[pl.ds(j * 16, 16)]
            plsc.store_scatter(gxb, [pos], gx, mask=m)
            plsc.store_scatter(gyb, [pos], gy, mask=m)
            plsc.store_scatter(gzb, [pos], gz, mask=m)
            return cur + plsc.all_reduce_population_count(m)

        cur0 = lax.broadcast(base, (16,))
        plsc.parallel_loop(0, _N // 16, carry=cur0, unroll=8)(scan_vreg)

        bofs = pl.multiple_of(base, 8)
        x0 = gxb[pl.ds(bofs, 16)]
        x1 = gxb[pl.ds(bofs + 16, 16)]
        x2 = gxb[pl.ds(bofs + 32, 16)]
        x3 = gxb[pl.ds(bofs + 48, 16)]
        y0 = gyb[pl.ds(bofs, 16)]
        y1 = gyb[pl.ds(bofs + 16, 16)]
        y2 = gyb[pl.ds(bofs + 32, 16)]
        y3 = gyb[pl.ds(bofs + 48, 16)]
        z0 = gzb[pl.ds(bofs, 16)]
        z1 = gzb[pl.ds(bofs + 16, 16)]
        z2 = gzb[pl.ds(bofs + 32, 16)]
        z3 = gzb[pl.ds(bofs + 48, 16)]
        mxv = lax.broadcast(jnp.sum((x0 + x1) + (x2 + x3)) * (1.0 / _NP), (16,))
        myv = lax.broadcast(jnp.sum((y0 + y1) + (y2 + y3)) * (1.0 / _NP), (16,))
        mzv = lax.broadcast(jnp.sum((z0 + z1) + (z2 + z3)) * (1.0 / _NP), (16,))

        def _nsq(a, b, c):
            da, db, dc = a - mxv, b - myv, c - mzv
            return da * da + db * db + dc * dc

        n0 = _nsq(x0, y0, z0)
        n1 = _nsq(x1, y1, z1)
        n2 = _nsq(x2, y2, z2)
        n3 = _nsq(x3, y3, z3)
        msqv = lax.broadcast(
            jnp.max(jnp.maximum(jnp.maximum(n0, n1), jnp.maximum(n2, n3))),
            (16,))
        zv = jnp.zeros((16,), jnp.float32)
        av = jnp.where(iota16 == 0, mxv,
                       jnp.where(iota16 == 1, myv,
                                 jnp.where(iota16 == 2, mzv,
                                           jnp.where(iota16 == 3, msqv, zv))))
        auxb[pl.ds(i * 16, 16)] = av

        oofs = pl.multiple_of(q * _NP, 64)
        pltpu.sync_copy(gxb.at[pl.ds(bofs, _NP)], outx_hbm.at[pl.ds(oofs, _NP)])
        pltpu.sync_copy(gyb.at[pl.ds(bofs, _NP)], outy_hbm.at[pl.ds(oofs, _NP)])
        pltpu.sync_copy(gzb.at[pl.ds(bofs, _NP)], outz_hbm.at[pl.ds(oofs, _NP)])
        return carry

    lax.fori_loop(0, _QPW, per_query, 0)
    pltpu.sync_copy(auxb, aux_hbm.at[pl.ds(woff, _QPW * 16)])


def _sc_select(d_flat, t16, data):
    nq = _B * _NC
    f = pl.kernel(
        _sc_body,
        mesh=plsc.VectorSubcoreMesh(core_axis_name="c", subcore_axis_name="s"),
        compiler_params=pltpu.CompilerParams(needs_layout_passes=False),
        out_type=[
            jax.ShapeDtypeStruct((nq * _NP,), jnp.float32),
            jax.ShapeDtypeStruct((nq * _NP,), jnp.float32),
            jax.ShapeDtypeStruct((nq * _NP,), jnp.float32),
            jax.ShapeDtypeStruct((nq * 16,), jnp.float32),
        ],
        scratch_types=[
            pltpu.VMEM((_N,), jnp.float32),
            pltpu.VMEM((_N,), jnp.float32),
            pltpu.VMEM((_N,), jnp.float32),
            pltpu.VMEM((_N,), jnp.float32),
            pltpu.VMEM((_QPW * 16,), jnp.float32),
            pltpu.VMEM((_QPW * _GW,), jnp.float32),
            pltpu.VMEM((_QPW * _GW,), jnp.float32),
            pltpu.VMEM((_QPW * _GW,), jnp.float32),
            pltpu.VMEM((_QPW * 16,), jnp.float32),
        ],
    )
    return f(d_flat, t16, data)


def _mm(a, b):
    return lax.dot_general(a, b, (((1,), (0,)), ((), ())),
                           preferred_element_type=jnp.float32)


def _fold_body(xr_ref, auxb_ref, auxf_ref, grid_ref,
               We1_ref, be1_ref, We2_ref, be2_ref, We3_ref, be3_ref,
               W1ac_ref, W1ag_ref, b1a_ref, W1b_ref, b1b_ref, W1c_ref,
               b1c_ref, W2ac_ref, W2af_ref, b2a_ref, W2b_ref, b2b_ref,
               W2c_ref, b2c_ref, out_ref):
    scale = jnp.sqrt(jnp.max(auxf_ref[:, 3:4]))
    inv = 1.0 / scale

    riota = lax.broadcasted_iota(jnp.int32, (_R, _PB), 0)
    piota = lax.broadcasted_iota(jnp.int32, (_R, _PB), 1)
    E = (riota // _NP == piota).astype(jnp.float32)
    riota2 = lax.broadcasted_iota(jnp.int32, (_R, _NP), 0)
    kiota = lax.broadcasted_iota(jnp.int32, (_R, _NP), 1)
    S = (riota2 % _NP == kiota).astype(jnp.float32)

    meanE = _mm(E, auxb_ref[:, 0:3])
    xx = (xr_ref[:, 0:1] - meanE[:, 0:1]) * inv
    yy = (xr_ref[:, 1:2] - meanE[:, 1:2]) * inv
    zz = (xr_ref[:, 2:3] - meanE[:, 2:3]) * inv
    We1 = We1_ref[...]
    h = jnp.maximum(xx * We1[0:1, :] + yy * We1[1:2, :] + zz * We1[2:3, :]
                    + be1_ref[...], 0.0)
    h = jnp.maximum(_mm(h, We2_ref[...]) + be2_ref[...], 0.0)
    h = _mm(h, We3_ref[...]) + be3_ref[...]
    code = jnp.max(h.reshape(_PB, _NP, 128), axis=1)

    g = grid_ref[...]
    W1ag = W1ag_ref[...]
    gW = g[:, 0:1] * W1ag[0:1, :] + g[:, 1:2] * W1ag[1:2, :]
    f = jnp.maximum(_mm(E, _mm(code, W1ac_ref[...])) + _mm(S, gW)
                    + b1a_ref[...], 0.0)
    f = jnp.maximum(_mm(f, W1b_ref[...]) + b1b_ref[...], 0.0)
    f3 = _mm(f, W1c_ref[...]) + b1c_ref[...]

    W2af = W2af_ref[...]
    o = jnp.maximum(_mm(E, _mm(code, W2ac_ref[...]))
                    + f3[:, 0:1] * W2af[0:1, :]
                    + f3[:, 1:2] * W2af[1:2, :]
                    + f3[:, 2:3] * W2af[2:3, :] + b2a_ref[...], 0.0)
    o = jnp.maximum(_mm(o, W2b_ref[...]) + b2b_ref[...], 0.0)
    o3 = _mm(o, W2c_ref[...]) + b2c_ref[...]
    out_ref[...] = o3 * scale + meanE


def _fold(xr, aux, weights):
    nprog = (_B * _NC) // _PB
    full = lambda shape: pl.BlockSpec(shape, lambda i: tuple(0 for _ in shape))
    in_specs = [
        pl.BlockSpec((_R, 3), lambda i: (i, 0)),
        pl.BlockSpec((_PB, 16), lambda i: (i, 0)),
        full(aux.shape),
        full((_NP, 2)),
    ] + [full(w.shape) for w in weights]
    return pl.pallas_call(
        _fold_body,
        grid=(nprog,),
        in_specs=in_specs,
        out_specs=pl.BlockSpec((_R, 3), lambda i: (i, 0)),
        out_shape=jax.ShapeDtypeStruct((_B * _NC * _NP, 3), jnp.float32),
    )(xr, aux, aux, jnp.asarray(_GRID), *weights)


def kernel(data, We1, be1, We2, be2, We3, be3, Wf1a, bf1a, Wf1b, bf1b, Wf1c,
           bf1c, Wf2a, bf2a, Wf2b, bf2b, Wf2c, bf2c):
    data3 = jnp.transpose(data, (1, 0, 2))
    cen = _fps(data3)
    perm = jax.random.permutation(jax.random.key(1), _NC)
    centers = jnp.transpose(cen, (1, 2, 0))[:, perm]
    d, t = _kthresh(data, centers)
    nq = _B * _NC
    t16 = jnp.broadcast_to(t.reshape(nq, 1), (nq, 16)).reshape(-1)
    outx, outy, outz, aux = _sc_select(d.reshape(nq, _N), t16,
                                       data.reshape(_B * 3, _N))
    xr = jnp.stack([outx, outy, outz], axis=-1)
    weights = [
        We1, be1.reshape(1, -1), We2, be2.reshape(1, -1), We3,
        be3.reshape(1, -1), Wf1a[:128], Wf1a[128:130], bf1a.reshape(1, -1),
        Wf1b, bf1b.reshape(1, -1), Wf1c, bf1c.reshape(1, -1), Wf2a[:128],
        Wf2a[128:131], bf2a.reshape(1, -1), Wf2b, bf2b.reshape(1, -1), Wf2c,
        bf2c.reshape(1, -1),
    ]
    out = _fold(xr, aux.reshape(nq, 16), weights)
    return out.reshape(_B, _NC, _NP, 3)
